# baseline (speedup 1.0000x reference)
import functools

import jax
import jax.numpy as jnp
from jax import lax
from jax.experimental import pallas as pl
from jax.experimental.pallas import tpu as pltpu
from jax.experimental.pallas import tpu_sc as plsc

_B = 16384
_L = 200
_TOT = _B * _L
_N = 1000000
_NW = 32
_PER_W = _TOT // _NW
_MINI = 4096
_CHUNK = 8192
_NCHUNK = (_PER_W - _MINI) // _CHUNK

_NSUB = 16
_SLICE = 62528
_NPAD = _SLICE * _NSUB
_NFILL = 8
_FILL = _SLICE // _NFILL


def _make_gather():
    info = plsc.get_sparse_core_info()
    nc = info.num_cores
    mesh = plsc.VectorSubcoreMesh(core_axis_name="c", subcore_axis_name="s")

    @functools.partial(
        pl.kernel,
        mesh=mesh,
        out_type=jax.ShapeDtypeStruct((_TOT,), jnp.float32),
        scratch_types=[
            pltpu.VMEM((_CHUNK,), jnp.int32),
            pltpu.VMEM((_CHUNK,), jnp.int32),
            pltpu.VMEM((_CHUNK,), jnp.float32),
            pltpu.VMEM((_CHUNK,), jnp.float32),
            pltpu.VMEM((_MINI,), jnp.int32),
            pltpu.VMEM((_MINI,), jnp.float32),
            pltpu.VMEM((_FILL,), jnp.float32),
            pltpu.VMEM((_FILL,), jnp.float32),
            pltpu.VMEM_SHARED((_NPAD,), jnp.float32),
            pltpu.SemaphoreType.DMA,
            pltpu.SemaphoreType.DMA,
            pltpu.SemaphoreType.DMA,
            pltpu.SemaphoreType.DMA,
            pltpu.SemaphoreType.DMA,
            pltpu.SemaphoreType.DMA,
            pltpu.SemaphoreType.DMA,
        ],
    )
    def gather_k(dist_hbm, idx_hbm, out_hbm,
                 idx0, idx1, out0, out1, mini_i, mini_o, bnc0, bnc1, tbl_sp,
                 si0, si1, sg0, sg1, so0, so1, sm):
        cid = lax.axis_index("c")
        sid = lax.axis_index("s")
        wid = sid * nc + cid
        base = wid * _PER_W

        idx = (idx0, idx1)
        out = (out0, out1)
        bnc = (bnc0, bnc1)
        sem_i = (si0, si1)
        sem_g = (sg0, sg1)
        sem_o = (so0, so1)

        def off(ci):
            return base + _MINI + ci * _CHUNK

        def load(ci):
            b = ci % 2
            pltpu.async_copy(
                idx_hbm.at[pl.ds(off(ci), _CHUNK)], idx[b], sem_i[b])

        def wait_load(ci):
            b = ci % 2
            pltpu.make_async_copy(
                idx_hbm.at[pl.ds(off(ci), _CHUNK)], idx[b], sem_i[b]).wait()

        def gather(ci):
            b = ci % 2
            pltpu.async_copy(tbl_sp.at[idx[b]], out[b], sem_g[b])

        def wait_gather(ci):
            b = ci % 2
            pltpu.make_async_copy(
                out_hbm.at[pl.ds(off(ci), _CHUNK)], out[b], sem_g[b]).wait()

        def store(ci):
            b = ci % 2
            pltpu.async_copy(
                out[b], out_hbm.at[pl.ds(off(ci), _CHUNK)], sem_o[b])

        def wait_store(ci):
            b = ci % 2
            pltpu.make_async_copy(
                out[b], out_hbm.at[pl.ds(off(ci), _CHUNK)], sem_o[b]).wait()

        s0 = sid * _SLICE

        def f0(k):
            return s0 + k * _FILL

        def fload(k):
            b = k % 2
            pltpu.async_copy(
                dist_hbm.at[pl.ds(f0(k), _FILL)], bnc[b], sem_g[b])

        def wait_fload(k):
            b = k % 2
            pltpu.make_async_copy(
                dist_hbm.at[pl.ds(f0(k), _FILL)], bnc[b], sem_g[b]).wait()

        def fstore(k):
            b = k % 2
            pltpu.async_copy(
                bnc[b], tbl_sp.at[pl.ds(f0(k), _FILL)], sem_o[b])

        def wait_fstore(k):
            b = k % 2
            pltpu.make_async_copy(
                bnc[b], tbl_sp.at[pl.ds(f0(k), _FILL)], sem_o[b]).wait()

        pltpu.async_copy(idx_hbm.at[pl.ds(base, _MINI)], mini_i, sm)
        fload(0)
        pltpu.make_async_copy(idx_hbm.at[pl.ds(base, _MINI)], mini_i, sm).wait()
        pltpu.async_copy(dist_hbm.at[mini_i], mini_o, sm)
        load(0)
        for k in range(_NFILL):
            wait_fload(k)
            if k >= 2:
                wait_fstore(k - 2)
            fstore(k)
            if k + 1 < _NFILL:
                fload(k + 1)
        wait_fstore(_NFILL - 2)
        wait_fstore(_NFILL - 1)
        pltpu.make_async_copy(
            out_hbm.at[pl.ds(base, _MINI)], mini_o, sm).wait()
        plsc.subcore_barrier()
        pltpu.async_copy(mini_o, out_hbm.at[pl.ds(base, _MINI)], sm)

        wait_load(0)
        gather(0)
        load(1)
        for ci in range(1, _NCHUNK):
            wait_load(ci)
            if ci >= 2:
                wait_store(ci - 2)
            gather(ci)
            wait_gather(ci - 1)
            store(ci - 1)
            if ci + 1 < _NCHUNK:
                load(ci + 1)
        wait_gather(_NCHUNK - 1)
        wait_store(_NCHUNK - 2)
        store(_NCHUNK - 1)
        wait_store(_NCHUNK - 1)
        pltpu.make_async_copy(
            mini_o, out_hbm.at[pl.ds(base, _MINI)], sm).wait()

    return gather_k


_gather = _make_gather()


def kernel(indices, distance):
    idx = indices.astype(jnp.int32).reshape(_TOT)
    dist_pad = jnp.pad(distance, (0, _NPAD - _N))
    out = _gather(dist_pad, idx)
    return out.reshape(_B, _L)


# speedup vs baseline: 1.0101x; 1.0101x over previous
import functools

import jax
import jax.numpy as jnp
from jax import lax
from jax.experimental import pallas as pl
from jax.experimental.pallas import tpu as pltpu
from jax.experimental.pallas import tpu_sc as plsc

_B = 16384
_L = 200
_TOT = _B * _L
_N = 1000000
_NW = 32
_PER_W = _TOT // _NW
_CHUNK = 12800
_NCHUNK = _PER_W // _CHUNK

_NSUB = 16
_SLICE = 62528
_NPAD = _SLICE * _NSUB
_NFILL = 8
_FILL = _SLICE // _NFILL


def _make_gather():
    info = plsc.get_sparse_core_info()
    nc = info.num_cores
    mesh = plsc.VectorSubcoreMesh(core_axis_name="c", subcore_axis_name="s")

    @functools.partial(
        pl.kernel,
        mesh=mesh,
        out_type=jax.ShapeDtypeStruct((_TOT,), jnp.float32),
        scratch_types=[
            pltpu.VMEM((_CHUNK,), jnp.int32),
            pltpu.VMEM((_CHUNK,), jnp.int32),
            pltpu.VMEM((_CHUNK,), jnp.float32),
            pltpu.VMEM((_CHUNK,), jnp.float32),
            pltpu.VMEM((_FILL,), jnp.float32),
            pltpu.VMEM((_FILL,), jnp.float32),
            pltpu.VMEM_SHARED((_NPAD,), jnp.float32),
            pltpu.SemaphoreType.DMA,
            pltpu.SemaphoreType.DMA,
            pltpu.SemaphoreType.DMA,
            pltpu.SemaphoreType.DMA,
            pltpu.SemaphoreType.DMA,
            pltpu.SemaphoreType.DMA,
        ],
    )
    def gather_k(dist_hbm, idx_hbm, out_hbm,
                 idx0, idx1, out0, out1, bnc0, bnc1, tbl_sp,
                 si0, si1, sg0, sg1, so0, so1):
        cid = lax.axis_index("c")
        sid = lax.axis_index("s")
        wid = sid * nc + cid
        base = wid * _PER_W

        idx = (idx0, idx1)
        out = (out0, out1)
        bnc = (bnc0, bnc1)
        sem_i = (si0, si1)
        sem_g = (sg0, sg1)
        sem_o = (so0, so1)

        def off(ci):
            return base + ci * _CHUNK

        def load(ci):
            b = ci % 2
            pltpu.async_copy(
                idx_hbm.at[pl.ds(off(ci), _CHUNK)], idx[b], sem_i[b])

        def wait_load(ci):
            b = ci % 2
            pltpu.make_async_copy(
                idx_hbm.at[pl.ds(off(ci), _CHUNK)], idx[b], sem_i[b]).wait()

        def gather(ci):
            b = ci % 2
            pltpu.async_copy(tbl_sp.at[idx[b]], out[b], sem_g[b])

        def wait_gather(ci):
            b = ci % 2
            pltpu.make_async_copy(
                out_hbm.at[pl.ds(off(ci), _CHUNK)], out[b], sem_g[b]).wait()

        def store(ci):
            b = ci % 2
            pltpu.async_copy(
                out[b], out_hbm.at[pl.ds(off(ci), _CHUNK)], sem_o[b])

        def wait_store(ci):
            b = ci % 2
            pltpu.make_async_copy(
                out[b], out_hbm.at[pl.ds(off(ci), _CHUNK)], sem_o[b]).wait()

        s0 = sid * _SLICE

        def f0(k):
            return s0 + k * _FILL

        def fload(k):
            b = k % 2
            pltpu.async_copy(
                dist_hbm.at[pl.ds(f0(k), _FILL)], bnc[b], sem_g[b])

        def wait_fload(k):
            b = k % 2
            pltpu.make_async_copy(
                dist_hbm.at[pl.ds(f0(k), _FILL)], bnc[b], sem_g[b]).wait()

        def fstore(k):
            b = k % 2
            pltpu.async_copy(
                bnc[b], tbl_sp.at[pl.ds(f0(k), _FILL)], sem_o[b])

        def wait_fstore(k):
            b = k % 2
            pltpu.make_async_copy(
                bnc[b], tbl_sp.at[pl.ds(f0(k), _FILL)], sem_o[b]).wait()

        load(0)
        fload(0)
        for k in range(_NFILL):
            wait_fload(k)
            if k >= 2:
                wait_fstore(k - 2)
            fstore(k)
            if k + 1 < _NFILL:
                fload(k + 1)
        wait_fstore(_NFILL - 2)
        wait_fstore(_NFILL - 1)
        plsc.subcore_barrier()

        wait_load(0)
        gather(0)
        load(1)
        for ci in range(1, _NCHUNK):
            wait_load(ci)
            if ci >= 2:
                wait_store(ci - 2)
            gather(ci)
            wait_gather(ci - 1)
            store(ci - 1)
            if ci + 1 < _NCHUNK:
                load(ci + 1)
        wait_gather(_NCHUNK - 1)
        wait_store(_NCHUNK - 2)
        store(_NCHUNK - 1)
        wait_store(_NCHUNK - 1)

    return gather_k


_gather = _make_gather()


def kernel(indices, distance):
    idx = indices.astype(jnp.int32).reshape(_TOT)
    dist_pad = jnp.pad(distance, (0, _NPAD - _N))
    out = _gather(dist_pad, idx)
    return out.reshape(_B, _L)
